# baseline (speedup 1.0000x reference)
import jax
import jax.numpy as jnp
from jax.experimental import pallas as pl
from jax.experimental.pallas import tpu as pltpu

_BIT = 8
_M = 30
_N = 20

_NCORE = 2
_BPC = 12
_NB = _NCORE * _BPC


def _shiftexp(x, scal_ref, b2):
    a2 = scal_ref[4]
    n2i = scal_ref[2]
    i0h = scal_ref[3]
    i0 = scal_ref[1]
    h = x * a2 + b2
    q = jnp.floor(h * n2i)
    ib = h + q * i0h + i0
    return ib * jnp.exp2(jnp.float32(_N) - q)


def _b2_from_pmax(pmax_ref):
    mxv = jnp.max(pmax_ref[...], axis=0, keepdims=True)
    b2v = mxv * jnp.float32(-0.71875)
    return b2v[0, 0, 0]


def _amax_body(scal_ref, x_ref, pmax_ref):
    inv_s = scal_ref[0]
    t = jnp.max(x_ref[...] * inv_s, axis=0, keepdims=True)
    u = jnp.max(t, axis=1, keepdims=True)
    pmax_ref[...] = jnp.broadcast_to(u.reshape(1, 1, 1), (1, 1, 128))


def _psum_body(scal_ref, x_ref, pmax_ref, psum_ref):
    b2 = _b2_from_pmax(pmax_ref)
    iexp = _shiftexp(x_ref[...], scal_ref, b2)
    t = jnp.sum(iexp, axis=0, keepdims=True)
    u = jnp.sum(t, axis=1, keepdims=True)
    psum_ref[...] = jnp.broadcast_to(u.reshape(1, 1, 1), (1, 1, 128))


def _out_body(scal_ref, x_ref, pmax_ref, psum_ref, o_ref):
    b2 = _b2_from_pmax(pmax_ref)
    sv = jnp.sum(psum_ref[...], axis=0, keepdims=True)
    cv = jnp.floor(jnp.float32(2.0 ** _M) / sv) * jnp.float32(2.0 ** (-_M))
    c = cv[0, 0, 0]
    iexp = _shiftexp(x_ref[...], scal_ref, b2)
    o_ref[...] = iexp * c


def kernel(x, scale):
    orig_shape = x.shape
    r = orig_shape[0] * orig_shape[1]
    cdim = orig_shape[2] * orig_shape[3]
    b = r // _NB
    x2 = x.reshape(r, cdim)

    s = scale.astype(jnp.float32)
    inv_s = 1.0 / s
    i0 = jnp.round(1.0 / s)
    scal = jnp.concatenate([
        inv_s,
        i0,
        -2.0 / i0,
        0.5 * i0,
        jnp.float32(0.71875) * inv_s,
    ]).astype(jnp.float32)

    smem_spec = pl.BlockSpec(memory_space=pltpu.SMEM)
    x_spec = pl.BlockSpec((b, cdim), lambda c, j: (c * _BPC + j, 0))
    part_out_spec = pl.BlockSpec((1, 1, 128), lambda c, j: (c * _BPC + j, 0, 0))
    part_in_spec = pl.BlockSpec((_NB, 1, 128), lambda c, j: (0, 0, 0))
    part_shape = jax.ShapeDtypeStruct((_NB, 1, 128), jnp.float32)
    params = pltpu.CompilerParams(
        dimension_semantics=("parallel", "arbitrary"))

    pmax = pl.pallas_call(
        _amax_body,
        grid=(_NCORE, _BPC),
        in_specs=[smem_spec, x_spec],
        out_specs=part_out_spec,
        out_shape=part_shape,
        compiler_params=params,
        name="qss_max",
    )(scal, x2)

    psum = pl.pallas_call(
        _psum_body,
        grid=(_NCORE, _BPC),
        in_specs=[smem_spec, x_spec, part_in_spec],
        out_specs=part_out_spec,
        out_shape=part_shape,
        compiler_params=params,
        name="qss_sum",
    )(scal, x2, pmax)

    out = pl.pallas_call(
        _out_body,
        grid=(_NCORE, _BPC),
        in_specs=[smem_spec, x_spec, part_in_spec, part_in_spec],
        out_specs=x_spec,
        out_shape=jax.ShapeDtypeStruct((r, cdim), jnp.float32),
        compiler_params=params,
        name="qss_out",
    )(scal, x2, pmax, psum)

    return out.reshape(orig_shape)


# speedup vs baseline: 1.7261x; 1.7261x over previous
import jax
import jax.numpy as jnp
from jax.experimental import pallas as pl
from jax.experimental.pallas import tpu as pltpu

_BIT = 8
_M = 30
_N = 20

_NCORE = 2
_BPC = 16
_NBLK = _NCORE * _BPC


def _shiftexp(x, scal_ref, b2):
    a2 = scal_ref[4]
    n2i = scal_ref[2]
    i0h = scal_ref[3]
    i0 = scal_ref[1]
    h = x * a2 + b2
    q = jnp.floor(h * n2i)
    ib = h + q * i0h + i0
    return ib * jnp.exp2(jnp.float32(_N) - q)


def _b2_from_pmax(pmax_ref):
    mxv = jnp.max(pmax_ref[...], axis=0, keepdims=True)
    b2v = mxv * jnp.float32(-0.71875)
    return b2v[0, 0, 0, 0]


def _amax_body(scal_ref, x_ref, pmax_ref):
    inv_s = scal_ref[0]
    m = jnp.max(x_ref[...] * inv_s)
    pmax_ref[...] = jnp.broadcast_to(m, (1, 1, 1, 128)).astype(jnp.float32)


def _psum_body(scal_ref, x_ref, pmax_ref, psum_ref):
    b2 = _b2_from_pmax(pmax_ref)
    iexp = _shiftexp(x_ref[...], scal_ref, b2)
    s = jnp.sum(iexp)
    psum_ref[...] = jnp.broadcast_to(s, (1, 1, 1, 128)).astype(jnp.float32)


def _out_body(scal_ref, x_ref, pmax_ref, psum_ref, o_ref):
    b2 = _b2_from_pmax(pmax_ref)
    sv = jnp.sum(psum_ref[...], axis=0, keepdims=True)
    cv = jnp.floor(jnp.float32(2.0 ** _M) / sv) * jnp.float32(2.0 ** (-_M))
    c = cv[0, 0, 0, 0]
    iexp = _shiftexp(x_ref[...], scal_ref, b2)
    o_ref[...] = iexp * c


def kernel(x, scale):
    d0, d1, d2, d3 = x.shape
    b0 = d0 // _NBLK

    s = scale.astype(jnp.float32)
    inv_s = 1.0 / s
    i0 = jnp.round(1.0 / s)
    scal = jnp.concatenate([
        inv_s,
        i0,
        -2.0 / i0,
        0.5 * i0,
        jnp.float32(0.71875) * inv_s,
    ]).astype(jnp.float32)

    smem_spec = pl.BlockSpec(memory_space=pltpu.SMEM)
    x_spec = pl.BlockSpec((b0, d1, d2, d3), lambda c, j: (c * _BPC + j, 0, 0, 0))
    part_out_spec = pl.BlockSpec(
        (1, 1, 1, 128), lambda c, j: (c * _BPC + j, 0, 0, 0))
    part_in_spec = pl.BlockSpec(
        (_NBLK, 1, 1, 128), lambda c, j: (0, 0, 0, 0))
    part_shape = jax.ShapeDtypeStruct((_NBLK, 1, 1, 128), jnp.float32)
    params = pltpu.CompilerParams(
        dimension_semantics=("parallel", "arbitrary"))

    pmax = pl.pallas_call(
        _amax_body,
        grid=(_NCORE, _BPC),
        in_specs=[smem_spec, x_spec],
        out_specs=part_out_spec,
        out_shape=part_shape,
        compiler_params=params,
        name="qss_max",
    )(scal, x)

    psum = pl.pallas_call(
        _psum_body,
        grid=(_NCORE, _BPC),
        in_specs=[smem_spec, x_spec, part_in_spec],
        out_specs=part_out_spec,
        out_shape=part_shape,
        compiler_params=params,
        name="qss_sum",
    )(scal, x, pmax)

    out = pl.pallas_call(
        _out_body,
        grid=(_NCORE, _BPC),
        in_specs=[smem_spec, x_spec, part_in_spec, part_in_spec],
        out_specs=x_spec,
        out_shape=jax.ShapeDtypeStruct((d0, d1, d2, d3), jnp.float32),
        compiler_params=params,
        name="qss_out",
    )(scal, x, pmax, psum)

    return out


# speedup vs baseline: 1.7459x; 1.0115x over previous
import jax
import jax.numpy as jnp
from jax.experimental import pallas as pl
from jax.experimental.pallas import tpu as pltpu

_BIT = 8
_M = 30
_N = 20

_NBLK = 32
_NEG_BIG = -3.0e38


def _shiftexp(x, scal_ref, b2):
    a2 = scal_ref[4]
    n2i = scal_ref[2]
    i0h = scal_ref[3]
    i0 = scal_ref[1]
    h = x * a2 + b2
    q = jnp.floor(h * n2i)
    ib = h + q * i0h + i0
    return ib * jnp.exp2(jnp.float32(_N) - q)


def _b2_scalar(smax_ref):
    b2v = smax_ref[...] * jnp.float32(-0.71875)
    return b2v[0, 0]


def _body(scal_ref, x_ref, o_ref, smax_ref, ssum_ref):
    p = pl.program_id(0)
    j = pl.program_id(1)

    @pl.when(p == 0)
    def _():
        m = jnp.max(x_ref[...] * scal_ref[0])
        mb = jnp.broadcast_to(m, (8, 128)).astype(jnp.float32)
        prev = jnp.where(j == 0, jnp.float32(_NEG_BIG), smax_ref[...])
        smax_ref[...] = jnp.maximum(prev, mb)

    @pl.when(p == 1)
    def _():
        iexp = _shiftexp(x_ref[...], scal_ref, _b2_scalar(smax_ref))
        s = jnp.sum(iexp)
        sb = jnp.broadcast_to(s, (8, 128)).astype(jnp.float32)
        prev = jnp.where(j == 0, jnp.float32(0.0), ssum_ref[...])
        ssum_ref[...] = prev + sb

    @pl.when(p == 2)
    def _():
        cv = (jnp.floor(jnp.float32(2.0 ** _M) / ssum_ref[...])
              * jnp.float32(2.0 ** (-_M)))
        c = cv[0, 0]
        iexp = _shiftexp(x_ref[...], scal_ref, _b2_scalar(smax_ref))
        o_ref[...] = iexp * c


def kernel(x, scale):
    d0, d1, d2, d3 = x.shape
    b0 = d0 // _NBLK

    s = scale.astype(jnp.float32)
    inv_s = 1.0 / s
    i0 = jnp.round(1.0 / s)
    scal = jnp.concatenate([
        inv_s,
        i0,
        -2.0 / i0,
        0.5 * i0,
        jnp.float32(0.71875) * inv_s,
    ]).astype(jnp.float32)

    out = pl.pallas_call(
        _body,
        grid=(3, _NBLK),
        in_specs=[
            pl.BlockSpec(memory_space=pltpu.SMEM),
            pl.BlockSpec((b0, d1, d2, d3), lambda p, j: (j, 0, 0, 0)),
        ],
        out_specs=pl.BlockSpec(
            (b0, d1, d2, d3),
            lambda p, j: (jnp.where(p == 2, j, 0), 0, 0, 0)),
        out_shape=jax.ShapeDtypeStruct((d0, d1, d2, d3), jnp.float32),
        scratch_shapes=[
            pltpu.VMEM((8, 128), jnp.float32),
            pltpu.VMEM((8, 128), jnp.float32),
        ],
        compiler_params=pltpu.CompilerParams(
            dimension_semantics=("arbitrary", "arbitrary")),
        name="qss_fused",
    )(scal, x)

    return out


# speedup vs baseline: 3.1435x; 1.8005x over previous
import jax
import jax.numpy as jnp
from jax.experimental import pallas as pl
from jax.experimental.pallas import tpu as pltpu

_BIT = 8
_M = 30
_N = 20

_NI = 12
_NH = 2
_NEG_BIG = -3.0e38


def _shiftexp(x, scal_ref, b2):
    a2 = scal_ref[4]
    n2i = scal_ref[2]
    i0h = scal_ref[3]
    i0 = scal_ref[1]
    h = x * a2 + b2
    q = jnp.floor(h * n2i)
    ib = h + q * i0h + i0
    return ib * jnp.exp2(jnp.float32(_N) - q)


def _b2_scalar(smax_ref):
    b2v = smax_ref[...] * jnp.float32(-0.71875)
    return b2v[0, 0]


def _body(scal_ref, x_ref, o_ref, smax_ref, ssum_ref):
    p = pl.program_id(0)
    i = pl.program_id(1)
    h = pl.program_id(2)
    first = jnp.logical_and(i == 0, h == 0)

    @pl.when(p == 0)
    def _():
        m = jnp.max(x_ref[...] * scal_ref[0])
        mb = jnp.broadcast_to(m, (8, 128)).astype(jnp.float32)
        prev = jnp.where(first, jnp.float32(_NEG_BIG), smax_ref[...])
        smax_ref[...] = jnp.maximum(prev, mb)

    @pl.when(p == 1)
    def _():
        iexp = _shiftexp(x_ref[...], scal_ref, _b2_scalar(smax_ref))
        s = jnp.sum(iexp)
        sb = jnp.broadcast_to(s, (8, 128)).astype(jnp.float32)
        prev = jnp.where(first, jnp.float32(0.0), ssum_ref[...])
        ssum_ref[...] = prev + sb

    @pl.when(p == 2)
    def _():
        cv = (jnp.floor(jnp.float32(2.0 ** _M) / ssum_ref[...])
              * jnp.float32(2.0 ** (-_M)))
        c = cv[0, 0]
        iexp = _shiftexp(x_ref[...], scal_ref, _b2_scalar(smax_ref))
        o_ref[...] = iexp * c


def kernel(x, scale):
    d0, d1, d2, d3 = x.shape
    xt = jnp.transpose(x, (1, 2, 0, 3))
    bi = d1 // _NI
    bh = d0 // _NH

    s = scale.astype(jnp.float32)
    inv_s = 1.0 / s
    i0 = jnp.round(1.0 / s)
    scal = jnp.concatenate([
        inv_s,
        i0,
        -2.0 / i0,
        0.5 * i0,
        jnp.float32(0.71875) * inv_s,
    ]).astype(jnp.float32)

    ot = pl.pallas_call(
        _body,
        grid=(3, _NI, _NH),
        in_specs=[
            pl.BlockSpec(memory_space=pltpu.SMEM),
            pl.BlockSpec((bi, d2, bh, d3), lambda p, i, h: (i, 0, h, 0)),
        ],
        out_specs=pl.BlockSpec(
            (bi, d2, bh, d3),
            lambda p, i, h: (jnp.where(p == 2, i, 0), 0,
                             jnp.where(p == 2, h, 0), 0)),
        out_shape=jax.ShapeDtypeStruct((d1, d2, d0, d3), jnp.float32),
        scratch_shapes=[
            pltpu.VMEM((8, 128), jnp.float32),
            pltpu.VMEM((8, 128), jnp.float32),
        ],
        compiler_params=pltpu.CompilerParams(
            dimension_semantics=("arbitrary", "arbitrary", "arbitrary")),
        name="qss_fused",
    )(scal, xt)

    return jnp.transpose(ot, (2, 0, 1, 3))


# speedup vs baseline: 3.9200x; 1.2470x over previous
import jax
import jax.numpy as jnp
from jax.experimental import pallas as pl
from jax.experimental.pallas import tpu as pltpu

_BIT = 8
_M = 30
_N = 20

_NI = 12
_NH = 2
_NEG_BIG = -3.0e38
_TOP = float((_N + 127) * (1 << 23))
_CLAMP = float(7 * (1 << 23))


def _iexp_raw(x, scal_ref, b2):
    bits = x * scal_ref[2] + b2
    bits = jnp.maximum(bits, jnp.float32(_CLAMP))
    return pltpu.bitcast(jnp.round(bits).astype(jnp.int32), jnp.float32)


def _b2_scalar(scal_ref, smax_ref):
    b2v = jnp.float32(_TOP) - smax_ref[...] * scal_ref[3]
    return b2v[0, 0]


def _body(scal_ref, x_ref, o_ref, smax_ref, ssum_ref):
    p = pl.program_id(0)
    i = pl.program_id(1)
    h = pl.program_id(2)
    first = jnp.logical_and(i == 0, h == 0)

    @pl.when(p == 0)
    def _():
        m = jnp.max(x_ref[...] * scal_ref[0])
        mb = jnp.broadcast_to(m, (8, 128)).astype(jnp.float32)
        prev = jnp.where(first, jnp.float32(_NEG_BIG), smax_ref[...])
        smax_ref[...] = jnp.maximum(prev, mb)

    @pl.when(p == 1)
    def _():
        iexp = _iexp_raw(x_ref[...], scal_ref, _b2_scalar(scal_ref, smax_ref))
        s = jnp.sum(iexp)
        sb = jnp.broadcast_to(s, (8, 128)).astype(jnp.float32)
        prev = jnp.where(first, jnp.float32(0.0), ssum_ref[...])
        ssum_ref[...] = prev + sb

    @pl.when(p == 2)
    def _():
        sv = ssum_ref[...] * scal_ref[1]
        cv = jnp.floor(jnp.float32(2.0 ** _M) / sv) * scal_ref[4]
        c = cv[0, 0]
        iexp = _iexp_raw(x_ref[...], scal_ref, _b2_scalar(scal_ref, smax_ref))
        o_ref[...] = iexp * c


def kernel(x, scale):
    d0, d1, d2, d3 = x.shape
    xt = jnp.transpose(x, (1, 2, 0, 3))
    bi = d1 // _NI
    bh = d0 // _NH

    s = scale.astype(jnp.float32)
    inv_s = 1.0 / s
    i0 = jnp.round(1.0 / s)
    g2 = jnp.float32(1.4375 * (1 << 23)) / i0
    scal = jnp.concatenate([
        inv_s,
        i0,
        g2 * inv_s,
        g2,
        jnp.float32(2.0 ** (-_M)) * i0,
    ]).astype(jnp.float32)

    ot = pl.pallas_call(
        _body,
        grid=(3, _NI, _NH),
        in_specs=[
            pl.BlockSpec(memory_space=pltpu.SMEM),
            pl.BlockSpec((bi, d2, bh, d3), lambda p, i, h: (i, 0, h, 0)),
        ],
        out_specs=pl.BlockSpec(
            (bi, d2, bh, d3),
            lambda p, i, h: (jnp.where(p == 2, i, 0), 0,
                             jnp.where(p == 2, h, 0), 0)),
        out_shape=jax.ShapeDtypeStruct((d1, d2, d0, d3), jnp.float32),
        scratch_shapes=[
            pltpu.VMEM((8, 128), jnp.float32),
            pltpu.VMEM((8, 128), jnp.float32),
        ],
        compiler_params=pltpu.CompilerParams(
            dimension_semantics=("arbitrary", "arbitrary", "arbitrary")),
        name="qss_fused",
    )(scal, xt)

    return jnp.transpose(ot, (2, 0, 1, 3))


# speedup vs baseline: 4.1620x; 1.0617x over previous
import jax
import jax.numpy as jnp
from jax.experimental import pallas as pl
from jax.experimental.pallas import tpu as pltpu

_BIT = 8
_M = 30
_N = 20

_NI = 12
_NH = 1
_NEG_BIG = -3.0e38
_TOP = float((_N + 127) * (1 << 23))
_CLAMP = float(7 * (1 << 23))


def _iexp_raw(x, scal_ref, b2):
    bits = x * scal_ref[2] + b2
    bits = jnp.maximum(bits, jnp.float32(_CLAMP))
    return pltpu.bitcast(jnp.round(bits).astype(jnp.int32), jnp.float32)


def _b2_scalar(scal_ref, smax_ref):
    b2v = jnp.float32(_TOP) - smax_ref[...] * scal_ref[3]
    return b2v[0, 0]


def _body(scal_ref, x_ref, o_ref, smax_ref, ssum_ref):
    p = pl.program_id(0)
    i = pl.program_id(1)
    h = pl.program_id(2)
    first = jnp.logical_and(i == 0, h == 0)

    @pl.when(p == 0)
    def _():
        m = jnp.max(x_ref[...] * scal_ref[0])
        mb = jnp.broadcast_to(m, (8, 128)).astype(jnp.float32)
        prev = jnp.where(first, jnp.float32(_NEG_BIG), smax_ref[...])
        smax_ref[...] = jnp.maximum(prev, mb)

    @pl.when(p == 1)
    def _():
        iexp = _iexp_raw(x_ref[...], scal_ref, _b2_scalar(scal_ref, smax_ref))
        s = jnp.sum(iexp)
        sb = jnp.broadcast_to(s, (8, 128)).astype(jnp.float32)
        prev = jnp.where(first, jnp.float32(0.0), ssum_ref[...])
        ssum_ref[...] = prev + sb

    @pl.when(p == 2)
    def _():
        sv = ssum_ref[...] * scal_ref[1]
        cv = jnp.floor(jnp.float32(2.0 ** _M) / sv) * scal_ref[4]
        c = cv[0, 0]
        iexp = _iexp_raw(x_ref[...], scal_ref, _b2_scalar(scal_ref, smax_ref))
        o_ref[...] = iexp * c


def kernel(x, scale):
    d0, d1, d2, d3 = x.shape
    xt = jnp.transpose(x, (1, 2, 0, 3))
    bi = d1 // _NI
    bh = d0 // _NH

    s = scale.astype(jnp.float32)
    inv_s = 1.0 / s
    i0 = jnp.round(1.0 / s)
    g2 = jnp.float32(1.4375 * (1 << 23)) / i0
    scal = jnp.concatenate([
        inv_s,
        i0,
        g2 * inv_s,
        g2,
        jnp.float32(2.0 ** (-_M)) * i0,
    ]).astype(jnp.float32)

    ot = pl.pallas_call(
        _body,
        grid=(3, _NI, _NH),
        in_specs=[
            pl.BlockSpec(memory_space=pltpu.SMEM),
            pl.BlockSpec((bi, d2, bh, d3), lambda p, i, h: (i, 0, h, 0)),
        ],
        out_specs=pl.BlockSpec(
            (bi, d2, bh, d3),
            lambda p, i, h: (jnp.where(p == 2, i, 0), 0,
                             jnp.where(p == 2, h, 0), 0)),
        out_shape=jax.ShapeDtypeStruct((d1, d2, d0, d3), jnp.float32),
        scratch_shapes=[
            pltpu.VMEM((8, 128), jnp.float32),
            pltpu.VMEM((8, 128), jnp.float32),
        ],
        compiler_params=pltpu.CompilerParams(
            dimension_semantics=("arbitrary", "arbitrary", "arbitrary"),
            vmem_limit_bytes=62 * 1024 * 1024),
        name="qss_fused",
    )(scal, xt)

    return jnp.transpose(ot, (2, 0, 1, 3))
